# baseline (speedup 1.0000x reference)
import jax
import jax.numpy as jnp
from jax import lax
from jax.experimental import pallas as pl
from jax.experimental.pallas import tpu as pltpu
from jax.experimental.pallas import tpu_sc as plsc

N = 10000
E = 320000
D = 128
NC = 2
NS = 16
L = 16
NPAD = 10240
CHUNK = 80
EPT = E // (NC * NS)
CPT = EPT // CHUNK
G = 5
NG = CPT // G
NSLOT = 2
NBUF = 4
AHEAD = 2
SLAG = NBUF - AHEAD
RPT = NPAD // NS
ZC = 80


def _sc_aggregate(with_deg: bool):
    out_type = [jax.ShapeDtypeStruct((NC, NPAD, D), jnp.float32)]
    if with_deg:
        out_type.append(jax.ShapeDtypeStruct((NC, 1, NPAD), jnp.float32))

    scratch = {
        "sidx": pltpu.VMEM((NSLOT * G * CHUNK,), jnp.int32),
        "didx": pltpu.VMEM((NSLOT * G * CHUNK,), jnp.int32),
        "rows": pltpu.VMEM((NBUF, CHUNK, D), jnp.float32),
        "fbuf": pltpu.VMEM((112,), jnp.float32),
        "acc": pltpu.VMEM_SHARED((NPAD, D), jnp.float32),
        "dacc": pltpu.VMEM_SHARED((NPAD,), jnp.float32),
        "gsem": pltpu.SemaphoreType.DMA((NBUF,)),
        "ssem": pltpu.SemaphoreType.DMA((NBUF,)),
        "dsem": pltpu.SemaphoreType.DMA((2,)),
        "isem": pltpu.SemaphoreType.DMA((NSLOT,)),
    }

    mesh = plsc.VectorSubcoreMesh(core_axis_name="c", subcore_axis_name="s")

    def body(x_hbm, src_hbm, dst_hbm, *outs, sidx, didx, rows, fbuf, acc,
             dacc, gsem, ssem, dsem, isem):
        if with_deg:
            nout, dout = outs
        else:
            (nout,) = outs

        c = lax.axis_index("c")
        s = lax.axis_index("s")
        wid = c * NS + s

        def stage(g, slot):
            base = wid * EPT + g * (G * CHUNK)
            sbase = slot * (G * CHUNK)
            pltpu.async_copy(src_hbm.at[pl.ds(base, G * CHUNK)],
                             sidx.at[pl.ds(sbase, G * CHUNK)], isem.at[slot])
            pltpu.async_copy(dst_hbm.at[pl.ds(base, G * CHUNK)],
                             didx.at[pl.ds(sbase, G * CHUNK)], isem.at[slot])

        def stage_wait(slot):
            for _ in range(2):
                pltpu.make_async_copy(src_hbm.at[pl.ds(0, G * CHUNK)],
                                      sidx.at[pl.ds(0, G * CHUNK)],
                                      isem.at[slot]).wait()

        def issue_gather(k):
            slot, j, b = (k // G) % NSLOT, k % G, k % NBUF
            pltpu.async_copy(
                x_hbm.at[sidx.at[pl.ds(slot * (G * CHUNK) + j * CHUNK,
                                       CHUNK)]],
                rows.at[b], gsem.at[b])

        def wait_gather(k):
            b = k % NBUF
            pltpu.make_async_copy(x_hbm.at[sidx.at[pl.ds(0, CHUNK)]],
                                  rows.at[b], gsem.at[b]).wait()

        def didx_ref(k):
            slot, j = (k // G) % NSLOT, k % G
            return didx.at[pl.ds(slot * (G * CHUNK) + j * CHUNK, CHUNK)]

        def issue_scatter(k):
            b = k % NBUF
            pltpu.async_copy(rows.at[b], acc.at[didx_ref(k)],
                             ssem.at[b], add=True)

        def wait_scatter(k):
            b = k % NBUF
            pltpu.make_async_copy(rows.at[b], acc.at[didx_ref(0)],
                                  ssem.at[b]).wait()

        def issue_deg(k):
            pltpu.async_copy(ones_src, dacc.at[didx_ref(k)],
                             dsem.at[k % 2], add=True)

        def wait_deg(k):
            pltpu.make_async_copy(ones_src, dacc.at[didx_ref(0)],
                                  dsem.at[k % 2]).wait()

        zero16 = jnp.zeros((L,), jnp.float32)
        for j in range(112 // L):
            fbuf[pl.ds(j * L, L)] = zero16

        stage(0, 0)
        stage(1, 1)

        @pl.loop(0, CHUNK)
        def _zrows(i):
            for j in range(D // L):
                rows[0, i, pl.ds(j * L, L)] = zero16

        zrows = rows.at[0, pl.ds(0, ZC)]
        zdeg = fbuf.at[pl.ds(0, ZC)]
        for k in range(RPT // ZC):
            pltpu.async_copy(zrows, acc.at[pl.ds(s * RPT + k * ZC, ZC)],
                             ssem.at[k % NBUF])
            if with_deg:
                pltpu.async_copy(zdeg, dacc.at[pl.ds(s * RPT + k * ZC, ZC)],
                                 dsem.at[k % 2])
        for k in range(RPT // ZC):
            pltpu.make_async_copy(zrows, acc.at[pl.ds(0, ZC)],
                                  ssem.at[k % NBUF]).wait()
            if with_deg:
                pltpu.make_async_copy(zdeg, dacc.at[pl.ds(0, ZC)],
                                      dsem.at[k % 2]).wait()

        if with_deg:
            one16 = jnp.ones((L,), jnp.float32)
            for j in range(112 // L):
                fbuf[pl.ds(j * L, L)] = one16
        ones_src = fbuf.at[pl.ds(0, CHUNK)]

        stage_wait(0)
        for k in range(AHEAD):
            issue_gather(k)

        plsc.subcore_barrier()

        @pl.loop(0, CPT)
        def _step(k):
            @pl.when(k >= SLAG)
            def _():
                wait_scatter(k - SLAG)

            if with_deg:
                @pl.when(k >= 1)
                def _():
                    wait_deg(k - 1)

            g = k // G

            @pl.when((k % G == 1) & (k > G) & (g + 1 < NG))
            def _():
                stage(g + 1, (g + 1) % NSLOT)

            @pl.when((k % G == G - AHEAD) & (k + AHEAD < CPT))
            def _():
                stage_wait((g + 1) % NSLOT)

            @pl.when(k + AHEAD < CPT)
            def _():
                issue_gather(k + AHEAD)

            wait_gather(k)
            issue_scatter(k)
            if with_deg:
                issue_deg(k)

        for k in range(CPT - SLAG, CPT):
            wait_scatter(k)
        if with_deg:
            wait_deg(CPT - 1)

        plsc.subcore_barrier()

        pltpu.sync_copy(acc.at[pl.ds(s * RPT, RPT)],
                        nout.at[c, pl.ds(s * RPT, RPT)])
        if with_deg:
            pltpu.sync_copy(dacc.at[pl.ds(s * RPT, RPT)],
                            dout.at[c, 0, pl.ds(s * RPT, RPT)])

    return pl.kernel(
        body,
        out_type=tuple(out_type) if with_deg else out_type[0],
        mesh=mesh,
        scratch_types=scratch,
    )


_sc_agg_deg = _sc_aggregate(with_deg=True)
_sc_agg = _sc_aggregate(with_deg=False)


BR = 2048


def _tc_body(pp, dd, xb, w, b, o):
    ssum = pp[0] + pp[1] + xb[...]
    deg = dd[0, 0] + dd[1, 0]
    inv = 1.0 / (deg + 1.0)
    h = ssum * inv[:, None]
    r = jnp.dot(h, w[...], preferred_element_type=jnp.float32) + b[...][None, :]
    o[...] = jnp.where(r > 0.0, r, jnp.exp(jnp.minimum(r, 0.0)) - 1.0)


def _tc_layer(np2, degp, xs, w, b):
    return pl.pallas_call(
        _tc_body,
        grid=(NPAD // BR,),
        in_specs=[
            pl.BlockSpec((NC, BR, D), lambda i: (0, i, 0)),
            pl.BlockSpec((NC, 1, BR), lambda i: (0, 0, i)),
            pl.BlockSpec((BR, D), lambda i: (i, 0)),
            pl.BlockSpec((D, D), lambda i: (0, 0)),
            pl.BlockSpec((D,), lambda i: (0,)),
        ],
        out_specs=pl.BlockSpec((BR, D), lambda i: (i, 0)),
        out_shape=jax.ShapeDtypeStruct((N, D), jnp.float32),
    )(np2, degp, xs, w, b)


def kernel(x, edge_index, W1, b1, W2, b2):
    src_flat = edge_index[0]
    dst_flat = edge_index[1]

    n1, degp = _sc_agg_deg(x, src_flat, dst_flat)
    h1 = _tc_layer(n1, degp, x, W1, b1)
    n2 = _sc_agg(h1, src_flat, dst_flat)
    h2 = _tc_layer(n2, degp, h1, W2, b2)
    return h2


# speedup vs baseline: 1.0625x; 1.0625x over previous
import jax
import jax.numpy as jnp
from jax import lax
from jax.experimental import pallas as pl
from jax.experimental.pallas import tpu as pltpu
from jax.experimental.pallas import tpu_sc as plsc

N = 10000
E = 320000
D = 128
NC = 2
NS = 16
L = 16
NPAD = 10240
CHUNK = 80
EPT = E // (NC * NS)
CPT = EPT // CHUNK
G = 5
NG = CPT // G
NSLOT = 2
NBUF = 4
AHEAD = 3
SLAG = NBUF - AHEAD
RPT = NPAD // NS
ZC = 80


def _sc_aggregate(with_deg: bool):
    out_type = [jax.ShapeDtypeStruct((NC, NPAD, D), jnp.float32)]
    if with_deg:
        out_type.append(jax.ShapeDtypeStruct((NC, 1, NPAD), jnp.float32))

    scratch = {
        "sidx": pltpu.VMEM((NSLOT * G * CHUNK,), jnp.int32),
        "didx": pltpu.VMEM((NSLOT * G * CHUNK,), jnp.int32),
        "rows": pltpu.VMEM((NBUF, CHUNK, D), jnp.float32),
        "fbuf": pltpu.VMEM((112,), jnp.float32),
        "acc": pltpu.VMEM_SHARED((NPAD, D), jnp.float32),
        "dacc": pltpu.VMEM_SHARED((NPAD,), jnp.float32),
        "gsem": pltpu.SemaphoreType.DMA((NBUF,)),
        "ssem": pltpu.SemaphoreType.DMA((NBUF,)),
        "dsem": pltpu.SemaphoreType.DMA((2,)),
        "isem": pltpu.SemaphoreType.DMA((NSLOT,)),
    }

    mesh = plsc.VectorSubcoreMesh(core_axis_name="c", subcore_axis_name="s")

    def body(x_hbm, src_hbm, dst_hbm, *outs, sidx, didx, rows, fbuf, acc,
             dacc, gsem, ssem, dsem, isem):
        if with_deg:
            nout, dout = outs
        else:
            (nout,) = outs

        c = lax.axis_index("c")
        s = lax.axis_index("s")
        wid = c * NS + s

        def stage(g, slot):
            base = wid * EPT + g * (G * CHUNK)
            sbase = slot * (G * CHUNK)
            pltpu.async_copy(src_hbm.at[pl.ds(base, G * CHUNK)],
                             sidx.at[pl.ds(sbase, G * CHUNK)], isem.at[slot])
            pltpu.async_copy(dst_hbm.at[pl.ds(base, G * CHUNK)],
                             didx.at[pl.ds(sbase, G * CHUNK)], isem.at[slot])

        def stage_wait(slot):
            for _ in range(2):
                pltpu.make_async_copy(src_hbm.at[pl.ds(0, G * CHUNK)],
                                      sidx.at[pl.ds(0, G * CHUNK)],
                                      isem.at[slot]).wait()

        def issue_gather(k):
            slot, j, b = (k // G) % NSLOT, k % G, k % NBUF
            pltpu.async_copy(
                x_hbm.at[sidx.at[pl.ds(slot * (G * CHUNK) + j * CHUNK,
                                       CHUNK)]],
                rows.at[b], gsem.at[b])

        def wait_gather(k):
            b = k % NBUF
            pltpu.make_async_copy(x_hbm.at[sidx.at[pl.ds(0, CHUNK)]],
                                  rows.at[b], gsem.at[b]).wait()

        def didx_ref(k):
            slot, j = (k // G) % NSLOT, k % G
            return didx.at[pl.ds(slot * (G * CHUNK) + j * CHUNK, CHUNK)]

        def issue_scatter(k):
            b = k % NBUF
            pltpu.async_copy(rows.at[b], acc.at[didx_ref(k)],
                             ssem.at[b], add=True)

        def wait_scatter(k):
            b = k % NBUF
            pltpu.make_async_copy(rows.at[b], acc.at[didx_ref(0)],
                                  ssem.at[b]).wait()

        def issue_deg(k):
            pltpu.async_copy(ones_src, dacc.at[didx_ref(k)],
                             dsem.at[k % 2], add=True)

        def wait_deg(k):
            pltpu.make_async_copy(ones_src, dacc.at[didx_ref(0)],
                                  dsem.at[k % 2]).wait()

        zero16 = jnp.zeros((L,), jnp.float32)
        for j in range(112 // L):
            fbuf[pl.ds(j * L, L)] = zero16

        stage(0, 0)
        stage(1, 1)

        @pl.loop(0, CHUNK)
        def _zrows(i):
            for j in range(D // L):
                rows[0, i, pl.ds(j * L, L)] = zero16

        zrows = rows.at[0, pl.ds(0, ZC)]
        zdeg = fbuf.at[pl.ds(0, ZC)]
        for k in range(RPT // ZC):
            pltpu.async_copy(zrows, acc.at[pl.ds(s * RPT + k * ZC, ZC)],
                             ssem.at[k % NBUF])
            if with_deg:
                pltpu.async_copy(zdeg, dacc.at[pl.ds(s * RPT + k * ZC, ZC)],
                                 dsem.at[k % 2])
        for k in range(RPT // ZC):
            pltpu.make_async_copy(zrows, acc.at[pl.ds(0, ZC)],
                                  ssem.at[k % NBUF]).wait()
            if with_deg:
                pltpu.make_async_copy(zdeg, dacc.at[pl.ds(0, ZC)],
                                      dsem.at[k % 2]).wait()

        if with_deg:
            one16 = jnp.ones((L,), jnp.float32)
            for j in range(112 // L):
                fbuf[pl.ds(j * L, L)] = one16
        ones_src = fbuf.at[pl.ds(0, CHUNK)]

        stage_wait(0)
        for k in range(AHEAD):
            issue_gather(k)

        plsc.subcore_barrier()

        @pl.loop(0, CPT)
        def _step(k):
            @pl.when(k >= SLAG)
            def _():
                wait_scatter(k - SLAG)

            if with_deg:
                @pl.when(k >= 1)
                def _():
                    wait_deg(k - 1)

            g = k // G

            @pl.when((k % G == 0) & (k > 0) & (g + 1 < NG))
            def _():
                stage(g + 1, (g + 1) % NSLOT)

            @pl.when((k % G == G - AHEAD) & (k + AHEAD < CPT))
            def _():
                stage_wait((g + 1) % NSLOT)

            @pl.when(k + AHEAD < CPT)
            def _():
                issue_gather(k + AHEAD)

            wait_gather(k)
            issue_scatter(k)
            if with_deg:
                issue_deg(k)

        for k in range(CPT - SLAG, CPT):
            wait_scatter(k)
        if with_deg:
            wait_deg(CPT - 1)

        plsc.subcore_barrier()

        pltpu.sync_copy(acc.at[pl.ds(s * RPT, RPT)],
                        nout.at[c, pl.ds(s * RPT, RPT)])
        if with_deg:
            pltpu.sync_copy(dacc.at[pl.ds(s * RPT, RPT)],
                            dout.at[c, 0, pl.ds(s * RPT, RPT)])

    return pl.kernel(
        body,
        out_type=tuple(out_type) if with_deg else out_type[0],
        mesh=mesh,
        scratch_types=scratch,
    )


_sc_agg_deg = _sc_aggregate(with_deg=True)
_sc_agg = _sc_aggregate(with_deg=False)


BR = 2048


def _tc_body(pp, dd, xb, w, b, o):
    ssum = pp[0] + pp[1] + xb[...]
    deg = dd[0, 0] + dd[1, 0]
    inv = 1.0 / (deg + 1.0)
    h = ssum * inv[:, None]
    r = jnp.dot(h, w[...], preferred_element_type=jnp.float32) + b[...][None, :]
    o[...] = jnp.where(r > 0.0, r, jnp.exp(jnp.minimum(r, 0.0)) - 1.0)


def _tc_layer(np2, degp, xs, w, b):
    return pl.pallas_call(
        _tc_body,
        grid=(NPAD // BR,),
        in_specs=[
            pl.BlockSpec((NC, BR, D), lambda i: (0, i, 0)),
            pl.BlockSpec((NC, 1, BR), lambda i: (0, 0, i)),
            pl.BlockSpec((BR, D), lambda i: (i, 0)),
            pl.BlockSpec((D, D), lambda i: (0, 0)),
            pl.BlockSpec((D,), lambda i: (0,)),
        ],
        out_specs=pl.BlockSpec((BR, D), lambda i: (i, 0)),
        out_shape=jax.ShapeDtypeStruct((N, D), jnp.float32),
    )(np2, degp, xs, w, b)


def kernel(x, edge_index, W1, b1, W2, b2):
    src_flat = edge_index[0]
    dst_flat = edge_index[1]

    n1, degp = _sc_agg_deg(x, src_flat, dst_flat)
    h1 = _tc_layer(n1, degp, x, W1, b1)
    n2 = _sc_agg(h1, src_flat, dst_flat)
    h2 = _tc_layer(n2, degp, h1, W2, b2)
    return h2


# speedup vs baseline: 1.1107x; 1.0454x over previous
import jax
import jax.numpy as jnp
from jax import lax
from jax.experimental import pallas as pl
from jax.experimental.pallas import tpu as pltpu
from jax.experimental.pallas import tpu_sc as plsc

N = 10000
E = 320000
D = 128
NC = 2
NS = 16
L = 16
NPAD = 10240
CHUNK = 80
EPT = E // (NC * NS)
CPT = EPT // CHUNK
G = 5
NG = CPT // G
NSLOT = 2
NBUF = 4
AHEAD = 3
SLAG = NBUF - AHEAD
RPT = NPAD // NS
ZC = 80


def _sc_aggregate(with_deg: bool):
    out_type = [jax.ShapeDtypeStruct((NC, NPAD, D), jnp.float32)]
    if with_deg:
        out_type.append(jax.ShapeDtypeStruct((NC, 1, NPAD), jnp.float32))

    scratch = {
        "sidx": pltpu.VMEM((NSLOT * G * CHUNK,), jnp.int32),
        "didx": pltpu.VMEM((NSLOT * G * CHUNK,), jnp.int32),
        "rows": pltpu.VMEM((NBUF, CHUNK, D), jnp.float32),
        "fbuf": pltpu.VMEM((112,), jnp.float32),
        "acc": pltpu.VMEM_SHARED((NPAD, D), jnp.float32),
        "dacc": pltpu.VMEM_SHARED((NPAD,), jnp.float32),
        "gsem": pltpu.SemaphoreType.DMA((NBUF,)),
        "ssem": pltpu.SemaphoreType.DMA((NBUF,)),
        "dsem": pltpu.SemaphoreType.DMA((2,)),
        "isem": pltpu.SemaphoreType.DMA((NSLOT,)),
    }

    mesh = plsc.VectorSubcoreMesh(core_axis_name="c", subcore_axis_name="s")

    def body(x_hbm, edge_hbm, *outs, sidx, didx, rows, fbuf, acc,
             dacc, gsem, ssem, dsem, isem):
        if with_deg:
            nout, dout = outs
        else:
            (nout,) = outs

        c = lax.axis_index("c")
        s = lax.axis_index("s")
        wid = c * NS + s

        def stage(g, slot):
            base = wid * EPT + g * (G * CHUNK)
            sbase = slot * (G * CHUNK)
            pltpu.async_copy(edge_hbm.at[pl.ds(base, G * CHUNK)],
                             sidx.at[pl.ds(sbase, G * CHUNK)], isem.at[slot])
            pltpu.async_copy(edge_hbm.at[pl.ds(E + base, G * CHUNK)],
                             didx.at[pl.ds(sbase, G * CHUNK)], isem.at[slot])

        def stage_wait(slot):
            for _ in range(2):
                pltpu.make_async_copy(edge_hbm.at[pl.ds(0, G * CHUNK)],
                                      sidx.at[pl.ds(0, G * CHUNK)],
                                      isem.at[slot]).wait()

        def issue_gather(k):
            slot, j, b = (k // G) % NSLOT, k % G, k % NBUF
            pltpu.async_copy(
                x_hbm.at[sidx.at[pl.ds(slot * (G * CHUNK) + j * CHUNK,
                                       CHUNK)]],
                rows.at[b], gsem.at[b])

        def wait_gather(k):
            b = k % NBUF
            pltpu.make_async_copy(x_hbm.at[sidx.at[pl.ds(0, CHUNK)]],
                                  rows.at[b], gsem.at[b]).wait()

        def didx_ref(k):
            slot, j = (k // G) % NSLOT, k % G
            return didx.at[pl.ds(slot * (G * CHUNK) + j * CHUNK, CHUNK)]

        def issue_scatter(k):
            b = k % NBUF
            pltpu.async_copy(rows.at[b], acc.at[didx_ref(k)],
                             ssem.at[b], add=True)

        def wait_scatter(k):
            b = k % NBUF
            pltpu.make_async_copy(rows.at[b], acc.at[didx_ref(0)],
                                  ssem.at[b]).wait()

        def issue_deg(k):
            pltpu.async_copy(ones_src, dacc.at[didx_ref(k)],
                             dsem.at[k % 2], add=True)

        def wait_deg(k):
            pltpu.make_async_copy(ones_src, dacc.at[didx_ref(0)],
                                  dsem.at[k % 2]).wait()

        zero16 = jnp.zeros((L,), jnp.float32)
        for j in range(112 // L):
            fbuf[pl.ds(j * L, L)] = zero16

        stage(0, 0)
        stage(1, 1)

        @pl.loop(0, CHUNK)
        def _zrows(i):
            for j in range(D // L):
                rows[0, i, pl.ds(j * L, L)] = zero16

        zrows = rows.at[0, pl.ds(0, ZC)]
        zdeg = fbuf.at[pl.ds(0, ZC)]
        for k in range(RPT // ZC):
            pltpu.async_copy(zrows, acc.at[pl.ds(s * RPT + k * ZC, ZC)],
                             ssem.at[k % NBUF])
            if with_deg:
                pltpu.async_copy(zdeg, dacc.at[pl.ds(s * RPT + k * ZC, ZC)],
                                 dsem.at[k % 2])
        for k in range(RPT // ZC):
            pltpu.make_async_copy(zrows, acc.at[pl.ds(0, ZC)],
                                  ssem.at[k % NBUF]).wait()
            if with_deg:
                pltpu.make_async_copy(zdeg, dacc.at[pl.ds(0, ZC)],
                                      dsem.at[k % 2]).wait()

        if with_deg:
            one16 = jnp.ones((L,), jnp.float32)
            for j in range(112 // L):
                fbuf[pl.ds(j * L, L)] = one16
        ones_src = fbuf.at[pl.ds(0, CHUNK)]

        stage_wait(0)
        for k in range(AHEAD):
            issue_gather(k)

        plsc.subcore_barrier()

        @pl.loop(0, CPT)
        def _step(k):
            @pl.when(k >= SLAG)
            def _():
                wait_scatter(k - SLAG)

            if with_deg:
                @pl.when(k >= 1)
                def _():
                    wait_deg(k - 1)

            g = k // G

            @pl.when((k % G == 0) & (k > 0) & (g + 1 < NG))
            def _():
                stage(g + 1, (g + 1) % NSLOT)

            @pl.when((k % G == G - AHEAD) & (k + AHEAD < CPT))
            def _():
                stage_wait((g + 1) % NSLOT)

            @pl.when(k + AHEAD < CPT)
            def _():
                issue_gather(k + AHEAD)

            wait_gather(k)
            issue_scatter(k)
            if with_deg:
                issue_deg(k)

        for k in range(CPT - SLAG, CPT):
            wait_scatter(k)
        if with_deg:
            wait_deg(CPT - 1)

        plsc.subcore_barrier()

        pltpu.sync_copy(acc.at[pl.ds(s * RPT, RPT)],
                        nout.at[c, pl.ds(s * RPT, RPT)])
        if with_deg:
            pltpu.sync_copy(dacc.at[pl.ds(s * RPT, RPT)],
                            dout.at[c, 0, pl.ds(s * RPT, RPT)])

    return pl.kernel(
        body,
        out_type=tuple(out_type) if with_deg else out_type[0],
        mesh=mesh,
        scratch_types=scratch,
    )


_sc_agg_deg = _sc_aggregate(with_deg=True)
_sc_agg = _sc_aggregate(with_deg=False)


BR = 2048


def _tc_body(pp, dd, xb, w, b, o):
    ssum = pp[0] + pp[1] + xb[...]
    deg = dd[0, 0] + dd[1, 0]
    inv = 1.0 / (deg + 1.0)
    h = ssum * inv[:, None]
    r = jnp.dot(h, w[...], preferred_element_type=jnp.float32) + b[...][None, :]
    o[...] = jnp.where(r > 0.0, r, jnp.exp(jnp.minimum(r, 0.0)) - 1.0)


def _tc_layer(np2, degp, xs, w, b):
    return pl.pallas_call(
        _tc_body,
        grid=(NPAD // BR,),
        in_specs=[
            pl.BlockSpec((NC, BR, D), lambda i: (0, i, 0)),
            pl.BlockSpec((NC, 1, BR), lambda i: (0, 0, i)),
            pl.BlockSpec((BR, D), lambda i: (i, 0)),
            pl.BlockSpec((D, D), lambda i: (0, 0)),
            pl.BlockSpec((D,), lambda i: (0,)),
        ],
        out_specs=pl.BlockSpec((BR, D), lambda i: (i, 0)),
        out_shape=jax.ShapeDtypeStruct((N, D), jnp.float32),
    )(np2, degp, xs, w, b)


def kernel(x, edge_index, W1, b1, W2, b2):
    edge_flat = edge_index.reshape(2 * E)

    n1, degp = _sc_agg_deg(x, edge_flat)
    h1 = _tc_layer(n1, degp, x, W1, b1)
    n2 = _sc_agg(h1, edge_flat)
    h2 = _tc_layer(n2, degp, h1, W2, b2)
    return h2
